# Initial kernel scaffold; baseline (speedup 1.0000x reference)
import jax
import jax.numpy as jnp
from jax.experimental import pallas as pl


def kernel(x, edge_index, W1, b1, W2, b2, W3, b3):
    raise NotImplementedError("write your pallas kernel here")


# speedup vs baseline: 19.3981x; 19.3981x over previous
import functools

import jax
import jax.numpy as jnp
from jax import lax
from jax.experimental import pallas as pl
from jax.experimental.pallas import tpu as pltpu
from jax.experimental.pallas import tpu_sc as plsc

_L = 16
_NC = 2
_NS = 16
_B = 128
_K = 20
_BN = 1000

_RPS = 6256
_NPAD = _RPS * _NS
_ZR = 391

_SC_PARAMS = pltpu.CompilerParams(use_tc_tiling_on_sc=False)


def _worker_span(w, nchunks, nworkers):
    q, r = nchunks // nworkers, nchunks % nworkers
    start = w * q + jnp.minimum(w, r)
    cnt = q + jnp.where(w < r, 1, 0).astype(jnp.int32)
    return start, start + cnt


def _zero_acc(zbuf, acc, s):
    zero16 = jnp.zeros((_L,), jnp.float32)

    @pl.loop(0, _ZR)
    def _(i):
        zbuf[i, :] = zero16

    @pl.loop(0, _RPS // _ZR)
    def _(t):
        pltpu.sync_copy(zbuf, acc.at[pl.ds(s * _RPS + t * _ZR, _ZR), :])


def _flush_acc(acc, out_hbm, c, s):
    pltpu.sync_copy(
        acc.at[pl.ds(s * _RPS, _RPS), :],
        out_hbm.at[c, pl.ds(s * _RPS, _RPS), :],
    )


def _sc_count(edge4, n_nodes):
    nchunks = edge4.shape[1]
    mesh = plsc.VectorSubcoreMesh(core_axis_name="c", subcore_axis_name="s")

    @functools.partial(
        pl.kernel,
        out_type=jax.ShapeDtypeStruct((_NC, _NPAD, _L), jnp.float32),
        mesh=mesh,
        compiler_params=_SC_PARAMS,
        scratch_types=[
            pltpu.VMEM((_K, _B), jnp.int32),
            pltpu.VMEM((_B, _L), jnp.float32),
            pltpu.VMEM((_ZR, _L), jnp.float32),
            pltpu.VMEM_SHARED((_NPAD, _L), jnp.float32),
        ],
    )
    def count_k(edge_hbm, out_hbm, didx, ones, zbuf, acc):
        c = lax.axis_index("c")
        s = lax.axis_index("s")
        one16 = jnp.ones((_L,), jnp.float32)

        @pl.loop(0, _B)
        def _(i):
            ones[i, :] = one16

        _zero_acc(zbuf, acc, s)
        plsc.subcore_barrier()

        w = s * _NC + c
        start, stop = _worker_span(w, nchunks, _NC * _NS)

        @pl.loop(start, stop)
        def _(ci):
            pltpu.sync_copy(edge_hbm.at[1, ci, :, :], didx)

            @pl.loop(0, _K)
            def _(k):
                pltpu.sync_copy(ones, acc.at[didx.at[k]], add=True)

        plsc.subcore_barrier()
        _flush_acc(acc, out_hbm, c, s)

    return count_k(edge4)


def _sc_scatter_pass(table, edge4, n_nodes, feature_split):
    nchunks = edge4.shape[1]
    mesh = plsc.VectorSubcoreMesh(core_axis_name="c", subcore_axis_name="s")

    @functools.partial(
        pl.kernel,
        out_type=jax.ShapeDtypeStruct((_NC, _NPAD, _L), jnp.float32),
        mesh=mesh,
        compiler_params=_SC_PARAMS,
        scratch_types=[
            pltpu.VMEM((_K, _B), jnp.int32),
            pltpu.VMEM((_K, _B), jnp.int32),
            pltpu.VMEM((_B, _L), jnp.float32),
            pltpu.VMEM((_ZR, _L), jnp.float32),
            pltpu.VMEM_SHARED((_NPAD, _L), jnp.float32),
            pltpu.SemaphoreType.DMA,
        ],
    )
    def pass_k(table_hbm, edge_hbm, out_hbm, sidx, didx, rows, zbuf, acc, gsem):
        c = lax.axis_index("c")
        s = lax.axis_index("s")

        _zero_acc(zbuf, acc, s)
        plsc.subcore_barrier()

        if feature_split:
            w = s
            start, stop = _worker_span(w, nchunks, _NS)
        else:
            w = s * _NC + c
            start, stop = _worker_span(w, nchunks, _NC * _NS)

        @pl.loop(start, stop)
        def _(ci):
            pltpu.sync_copy(edge_hbm.at[0, ci, :, :], sidx)
            pltpu.sync_copy(edge_hbm.at[1, ci, :, :], didx)
            if feature_split:
                off = jnp.full((_L,), c * n_nodes, jnp.int32)

                @pl.loop(0, _K)
                def _(k):
                    for i in range(_B // _L):
                        sl = pl.ds(i * _L, _L)
                        sidx[k, sl] = sidx[k, sl] + off

            @pl.loop(0, _K)
            def _(k):
                pltpu.async_copy(table_hbm.at[sidx.at[k]], rows, gsem).wait()
                pltpu.sync_copy(rows, acc.at[didx.at[k]], add=True)

        plsc.subcore_barrier()
        _flush_acc(acc, out_hbm, c, s)

    return pass_k(table, edge4)


def _tc_prep(degp, x):
    n, in_dim = x.shape

    def body(degp_ref, x_ref, dinv_ref, g1_ref):
        d = degp_ref[0, :, 0] + degp_ref[1, :, 0] + 1.0
        di = lax.rsqrt(d)
        dinv_ref[...] = di[:, None]
        xd = x_ref[...] * di[:, None]
        g1_ref[...] = jnp.concatenate(
            [xd, jnp.zeros((_BN, _L - in_dim), jnp.float32)], axis=1
        )

    return pl.pallas_call(
        body,
        grid=(n // _BN,),
        in_specs=[
            pl.BlockSpec((_NC, _BN, _L), lambda j: (0, j, 0)),
            pl.BlockSpec((_BN, in_dim), lambda j: (j, 0)),
        ],
        out_specs=[
            pl.BlockSpec((_BN, 1), lambda j: (j, 0)),
            pl.BlockSpec((_BN, _L), lambda j: (j, 0)),
        ],
        out_shape=[
            jax.ShapeDtypeStruct((n, 1), jnp.float32),
            jax.ShapeDtypeStruct((n, _L), jnp.float32),
        ],
    )(degp, x)


def _tc_layer1(pB, g1, dinv, W1, b1):
    n = g1.shape[0]
    in_dim, hid = W1.shape

    def body(p_ref, g1_ref, dinv_ref, w_ref, b_ref, t2_ref):
        di = dinv_ref[...]
        sacc = (p_ref[0] + p_ref[1] + g1_ref[...]) * di
        s3 = sacc[:, :in_dim]
        h1 = jax.lax.dot_general(
            s3, w_ref[...], (((1,), (0,)), ((), ())),
            preferred_element_type=jnp.float32,
        )
        h1 = jnp.maximum(h1 + b_ref[...], 0.0)
        g2 = h1 * di
        t2_ref[0] = g2[:, :_L]
        t2_ref[1] = g2[:, _L:]

    return pl.pallas_call(
        body,
        grid=(n // _BN,),
        in_specs=[
            pl.BlockSpec((_NC, _BN, _L), lambda j: (0, j, 0)),
            pl.BlockSpec((_BN, _L), lambda j: (j, 0)),
            pl.BlockSpec((_BN, 1), lambda j: (j, 0)),
            pl.BlockSpec((in_dim, hid), lambda j: (0, 0)),
            pl.BlockSpec((1, hid), lambda j: (0, 0)),
        ],
        out_specs=[pl.BlockSpec((_NC, _BN, _L), lambda j: (0, j, 0))],
        out_shape=[jax.ShapeDtypeStruct((_NC, n, _L), jnp.float32)],
    )(pB, g1, dinv, W1, b1)[0]


def _tc_layer2(pC, t2, dinv, W2, b2, W3):
    n = dinv.shape[0]
    hid = W2.shape[0]
    lat = W3.shape[1]

    def body(p_ref, t2_ref, dinv_ref, w2_ref, b2_ref, w3_ref, g3_ref):
        di = dinv_ref[...]
        o2a = (p_ref[0] + t2_ref[0]) * di
        o2b = (p_ref[1] + t2_ref[1]) * di
        out2 = jnp.concatenate([o2a, o2b], axis=1)
        h2 = jax.lax.dot_general(
            out2, w2_ref[...], (((1,), (0,)), ((), ())),
            preferred_element_type=jnp.float32,
        )
        h2 = jnp.maximum(h2 + b2_ref[...], 0.0)
        t = jax.lax.dot_general(
            h2, w3_ref[...], (((1,), (0,)), ((), ())),
            preferred_element_type=jnp.float32,
        )
        g3_ref[...] = jnp.concatenate(
            [t * di, jnp.zeros((_BN, _L - lat), jnp.float32)], axis=1
        )

    return pl.pallas_call(
        body,
        grid=(n // _BN,),
        in_specs=[
            pl.BlockSpec((_NC, _BN, _L), lambda j: (0, j, 0)),
            pl.BlockSpec((_NC, _BN, _L), lambda j: (0, j, 0)),
            pl.BlockSpec((_BN, 1), lambda j: (j, 0)),
            pl.BlockSpec((hid, hid), lambda j: (0, 0)),
            pl.BlockSpec((1, hid), lambda j: (0, 0)),
            pl.BlockSpec((hid, lat), lambda j: (0, 0)),
        ],
        out_specs=[pl.BlockSpec((_BN, _L), lambda j: (j, 0))],
        out_shape=[jax.ShapeDtypeStruct((n, _L), jnp.float32)],
    )(pC, t2, dinv, W2, b2, W3)[0]


def _tc_layer3(pD, g3, dinv, b3):
    n = dinv.shape[0]
    lat = b3.shape[1]

    def body(p_ref, g3_ref, dinv_ref, b_ref, out_ref):
        sacc = (p_ref[0] + p_ref[1] + g3_ref[...]) * dinv_ref[...]
        out_ref[...] = sacc[:, :lat] + b_ref[...]

    return pl.pallas_call(
        body,
        grid=(n // _BN,),
        in_specs=[
            pl.BlockSpec((_NC, _BN, _L), lambda j: (0, j, 0)),
            pl.BlockSpec((_BN, _L), lambda j: (j, 0)),
            pl.BlockSpec((_BN, 1), lambda j: (j, 0)),
            pl.BlockSpec((1, lat), lambda j: (0, 0)),
        ],
        out_specs=[pl.BlockSpec((_BN, lat), lambda j: (j, 0))],
        out_shape=[jax.ShapeDtypeStruct((n, lat), jnp.float32)],
    )(pD, g3, dinv, b3)[0]


def kernel(x, edge_index, W1, b1, W2, b2, W3, b3):
    n = x.shape[0]
    e = edge_index.shape[1]
    edge4 = edge_index.reshape(2, e // (_K * _B), _K, _B)

    degp = _sc_count(edge4, n)
    dinv, g1 = _tc_prep(degp, x)
    pB = _sc_scatter_pass(g1, edge4, n, False)
    t2 = _tc_layer1(pB, g1, dinv, W1, b1.reshape(1, -1))
    pC = _sc_scatter_pass(t2.reshape(2 * n, _L), edge4, n, True)
    g3 = _tc_layer2(pC, t2, dinv, W2, b2.reshape(1, -1), W3)
    pD = _sc_scatter_pass(g3, edge4, n, False)
    out = _tc_layer3(pD, g3, dinv, b3.reshape(1, -1))
    return out


# speedup vs baseline: 31.5334x; 1.6256x over previous
import functools

import jax
import jax.numpy as jnp
from jax import lax
from jax.experimental import pallas as pl
from jax.experimental.pallas import tpu as pltpu
from jax.experimental.pallas import tpu_sc as plsc

_L = 16
_NC = 2
_NS = 16
_B = 128
_K = 20
_D = 4
_BN = 1000

_RPS = 6256
_NPAD = _RPS * _NS
_ZR = 391

_SC_PARAMS = pltpu.CompilerParams(use_tc_tiling_on_sc=False)


def _worker_span(w, nchunks, nworkers):
    q, r = nchunks // nworkers, nchunks % nworkers
    start = w * q + jnp.minimum(w, r)
    cnt = q + jnp.where(w < r, 1, 0).astype(jnp.int32)
    return start, start + cnt


def _zero_acc(zbuf, acc, s):
    zero16 = jnp.zeros((_L,), jnp.float32)

    @pl.loop(0, _ZR)
    def _(i):
        zbuf[i, :] = zero16

    @pl.loop(0, _RPS // _ZR)
    def _(t):
        pltpu.sync_copy(zbuf, acc.at[pl.ds(s * _RPS + t * _ZR, _ZR), :])


def _flush_acc(acc, out_hbm, c, s):
    pltpu.sync_copy(
        acc.at[pl.ds(s * _RPS, _RPS), :],
        out_hbm.at[c, pl.ds(s * _RPS, _RPS), :],
    )


def _sc_count(edge4, n_nodes):
    nchunks = edge4.shape[1]
    mesh = plsc.VectorSubcoreMesh(core_axis_name="c", subcore_axis_name="s")

    @functools.partial(
        pl.kernel,
        out_type=jax.ShapeDtypeStruct((_NC, _NPAD, _L), jnp.float32),
        mesh=mesh,
        compiler_params=_SC_PARAMS,
        scratch_types=[
            pltpu.VMEM((_K, _B), jnp.int32),
            pltpu.VMEM((_B, _L), jnp.float32),
            pltpu.VMEM((_ZR, _L), jnp.float32),
            pltpu.VMEM_SHARED((_NPAD, _L), jnp.float32),
        ],
    )
    def count_k(edge_hbm, out_hbm, didx, ones, zbuf, acc):
        c = lax.axis_index("c")
        s = lax.axis_index("s")
        one16 = jnp.ones((_L,), jnp.float32)

        @pl.loop(0, _B)
        def _(i):
            ones[i, :] = one16

        _zero_acc(zbuf, acc, s)
        plsc.subcore_barrier()

        w = s * _NC + c
        start, stop = _worker_span(w, nchunks, _NC * _NS)

        @pl.loop(start, stop)
        def _(ci):
            pltpu.sync_copy(edge_hbm.at[1, ci, :, :], didx)

            @pl.loop(0, _K)
            def _(k):
                pltpu.sync_copy(ones, acc.at[didx.at[k]], add=True)

        plsc.subcore_barrier()
        _flush_acc(acc, out_hbm, c, s)

    return count_k(edge4)


def _sc_scatter_pass(table, edge4, n_nodes, feature_split):
    nchunks = edge4.shape[1]
    mesh = plsc.VectorSubcoreMesh(core_axis_name="c", subcore_axis_name="s")

    @functools.partial(
        pl.kernel,
        out_type=jax.ShapeDtypeStruct((_NC, _NPAD, _L), jnp.float32),
        mesh=mesh,
        compiler_params=_SC_PARAMS,
        scratch_types=[
            pltpu.VMEM((_K, _B), jnp.int32),
            pltpu.VMEM((_K, _B), jnp.int32),
            pltpu.VMEM((_D, _B, _L), jnp.float32),
            pltpu.VMEM((_ZR, _L), jnp.float32),
            pltpu.VMEM_SHARED((_NPAD, _L), jnp.float32),
            pltpu.SemaphoreType.DMA((_D,)),
        ],
    )
    def pass_k(table_hbm, edge_hbm, out_hbm, sidx, didx, rows, zbuf, acc, gsem):
        c = lax.axis_index("c")
        s = lax.axis_index("s")

        _zero_acc(zbuf, acc, s)
        plsc.subcore_barrier()

        if feature_split:
            w = s
            start, stop = _worker_span(w, nchunks, _NS)
        else:
            w = s * _NC + c
            start, stop = _worker_span(w, nchunks, _NC * _NS)

        @pl.loop(start, stop)
        def _(ci):
            pltpu.sync_copy(edge_hbm.at[0, ci, :, :], sidx)
            pltpu.sync_copy(edge_hbm.at[1, ci, :, :], didx)
            if feature_split:
                off = jnp.full((_L,), c * n_nodes, jnp.int32)

                @pl.loop(0, _K)
                def _(k):
                    for i in range(_B // _L):
                        sl = pl.ds(i * _L, _L)
                        sidx[k, sl] = sidx[k, sl] + off

            for k in range(_D):
                pltpu.async_copy(table_hbm.at[sidx.at[k]], rows.at[k], gsem.at[k])

            @pl.loop(0, _K)
            def _(k):
                slot = lax.rem(k, _D)
                pltpu.make_async_copy(
                    table_hbm.at[sidx.at[k]], rows.at[slot], gsem.at[slot]
                ).wait()
                pltpu.sync_copy(rows.at[slot], acc.at[didx.at[k]], add=True)

                @pl.when(k + _D < _K)
                def _():
                    pltpu.async_copy(
                        table_hbm.at[sidx.at[k + _D]], rows.at[slot], gsem.at[slot]
                    )

        plsc.subcore_barrier()
        _flush_acc(acc, out_hbm, c, s)

    return pass_k(table, edge4)


def _tc_prep(degp, x):
    n, in_dim = x.shape

    def body(degp_ref, x_ref, dinv_ref, g1_ref):
        d = degp_ref[0, :, 0] + degp_ref[1, :, 0] + 1.0
        di = lax.rsqrt(d)
        dinv_ref[...] = di[:, None]
        xd = x_ref[...] * di[:, None]
        g1_ref[...] = jnp.concatenate(
            [xd, jnp.zeros((_BN, _L - in_dim), jnp.float32)], axis=1
        )

    return pl.pallas_call(
        body,
        grid=(n // _BN,),
        in_specs=[
            pl.BlockSpec((_NC, _BN, _L), lambda j: (0, j, 0)),
            pl.BlockSpec((_BN, in_dim), lambda j: (j, 0)),
        ],
        out_specs=[
            pl.BlockSpec((_BN, 1), lambda j: (j, 0)),
            pl.BlockSpec((_BN, _L), lambda j: (j, 0)),
        ],
        out_shape=[
            jax.ShapeDtypeStruct((n, 1), jnp.float32),
            jax.ShapeDtypeStruct((n, _L), jnp.float32),
        ],
    )(degp, x)


def _tc_layer1(pB, g1, dinv, W1, b1):
    n = g1.shape[0]
    in_dim, hid = W1.shape

    def body(p_ref, g1_ref, dinv_ref, w_ref, b_ref, t2_ref):
        di = dinv_ref[...]
        sacc = (p_ref[0] + p_ref[1] + g1_ref[...]) * di
        s3 = sacc[:, :in_dim]
        h1 = jax.lax.dot_general(
            s3, w_ref[...], (((1,), (0,)), ((), ())),
            preferred_element_type=jnp.float32,
        )
        h1 = jnp.maximum(h1 + b_ref[...], 0.0)
        g2 = h1 * di
        t2_ref[0] = g2[:, :_L]
        t2_ref[1] = g2[:, _L:]

    return pl.pallas_call(
        body,
        grid=(n // _BN,),
        in_specs=[
            pl.BlockSpec((_NC, _BN, _L), lambda j: (0, j, 0)),
            pl.BlockSpec((_BN, _L), lambda j: (j, 0)),
            pl.BlockSpec((_BN, 1), lambda j: (j, 0)),
            pl.BlockSpec((in_dim, hid), lambda j: (0, 0)),
            pl.BlockSpec((1, hid), lambda j: (0, 0)),
        ],
        out_specs=[pl.BlockSpec((_NC, _BN, _L), lambda j: (0, j, 0))],
        out_shape=[jax.ShapeDtypeStruct((_NC, n, _L), jnp.float32)],
    )(pB, g1, dinv, W1, b1)[0]


def _tc_layer2(pC, t2, dinv, W2, b2, W3):
    n = dinv.shape[0]
    hid = W2.shape[0]
    lat = W3.shape[1]

    def body(p_ref, t2_ref, dinv_ref, w2_ref, b2_ref, w3_ref, g3_ref):
        di = dinv_ref[...]
        o2a = (p_ref[0] + t2_ref[0]) * di
        o2b = (p_ref[1] + t2_ref[1]) * di
        out2 = jnp.concatenate([o2a, o2b], axis=1)
        h2 = jax.lax.dot_general(
            out2, w2_ref[...], (((1,), (0,)), ((), ())),
            preferred_element_type=jnp.float32,
        )
        h2 = jnp.maximum(h2 + b2_ref[...], 0.0)
        t = jax.lax.dot_general(
            h2, w3_ref[...], (((1,), (0,)), ((), ())),
            preferred_element_type=jnp.float32,
        )
        g3_ref[...] = jnp.concatenate(
            [t * di, jnp.zeros((_BN, _L - lat), jnp.float32)], axis=1
        )

    return pl.pallas_call(
        body,
        grid=(n // _BN,),
        in_specs=[
            pl.BlockSpec((_NC, _BN, _L), lambda j: (0, j, 0)),
            pl.BlockSpec((_NC, _BN, _L), lambda j: (0, j, 0)),
            pl.BlockSpec((_BN, 1), lambda j: (j, 0)),
            pl.BlockSpec((hid, hid), lambda j: (0, 0)),
            pl.BlockSpec((1, hid), lambda j: (0, 0)),
            pl.BlockSpec((hid, lat), lambda j: (0, 0)),
        ],
        out_specs=[pl.BlockSpec((_BN, _L), lambda j: (j, 0))],
        out_shape=[jax.ShapeDtypeStruct((n, _L), jnp.float32)],
    )(pC, t2, dinv, W2, b2, W3)[0]


def _tc_layer3(pD, g3, dinv, b3):
    n = dinv.shape[0]
    lat = b3.shape[1]

    def body(p_ref, g3_ref, dinv_ref, b_ref, out_ref):
        sacc = (p_ref[0] + p_ref[1] + g3_ref[...]) * dinv_ref[...]
        out_ref[...] = sacc[:, :lat] + b_ref[...]

    return pl.pallas_call(
        body,
        grid=(n // _BN,),
        in_specs=[
            pl.BlockSpec((_NC, _BN, _L), lambda j: (0, j, 0)),
            pl.BlockSpec((_BN, _L), lambda j: (j, 0)),
            pl.BlockSpec((_BN, 1), lambda j: (j, 0)),
            pl.BlockSpec((1, lat), lambda j: (0, 0)),
        ],
        out_specs=[pl.BlockSpec((_BN, lat), lambda j: (j, 0))],
        out_shape=[jax.ShapeDtypeStruct((n, lat), jnp.float32)],
    )(pD, g3, dinv, b3)[0]


def kernel(x, edge_index, W1, b1, W2, b2, W3, b3):
    n = x.shape[0]
    e = edge_index.shape[1]
    edge4 = edge_index.reshape(2, e // (_K * _B), _K, _B)

    degp = _sc_count(edge4, n)
    dinv, g1 = _tc_prep(degp, x)
    pB = _sc_scatter_pass(g1, edge4, n, False)
    t2 = _tc_layer1(pB, g1, dinv, W1, b1.reshape(1, -1))
    pC = _sc_scatter_pass(t2.reshape(2 * n, _L), edge4, n, True)
    g3 = _tc_layer2(pC, t2, dinv, W2, b2.reshape(1, -1), W3)
    pD = _sc_scatter_pass(g3, edge4, n, False)
    out = _tc_layer3(pD, g3, dinv, b3.reshape(1, -1))
    return out


# speedup vs baseline: 34.4254x; 1.0917x over previous
import functools

import jax
import jax.numpy as jnp
from jax import lax
from jax.experimental import pallas as pl
from jax.experimental.pallas import tpu as pltpu
from jax.experimental.pallas import tpu_sc as plsc

_L = 16
_NC = 2
_NS = 16
_B = 128
_K = 20
_D = 4
_BN = 5000

_RPS = 6256
_NPAD = _RPS * _NS
_ZR = 391

_SC_PARAMS = pltpu.CompilerParams(use_tc_tiling_on_sc=False)


def _worker_span(w, nchunks, nworkers):
    q, r = nchunks // nworkers, nchunks % nworkers
    start = w * q + jnp.minimum(w, r)
    cnt = q + jnp.where(w < r, 1, 0).astype(jnp.int32)
    return start, start + cnt


def _zero_acc(zbuf, acc, s):
    zero16 = jnp.zeros((_L,), jnp.float32)

    @pl.loop(0, _ZR)
    def _(i):
        zbuf[i, :] = zero16

    @pl.loop(0, _RPS // _ZR)
    def _(t):
        pltpu.sync_copy(zbuf, acc.at[pl.ds(s * _RPS + t * _ZR, _ZR), :])


def _flush_acc(acc, out_hbm, c, s):
    pltpu.sync_copy(
        acc.at[pl.ds(s * _RPS, _RPS), :],
        out_hbm.at[c, pl.ds(s * _RPS, _RPS), :],
    )


def _sc_count(edge4, n_nodes):
    nchunks = edge4.shape[1]
    mesh = plsc.VectorSubcoreMesh(core_axis_name="c", subcore_axis_name="s")

    @functools.partial(
        pl.kernel,
        out_type=jax.ShapeDtypeStruct((_NC, _NPAD, _L), jnp.float32),
        mesh=mesh,
        compiler_params=_SC_PARAMS,
        scratch_types=[
            pltpu.VMEM((_K, _B), jnp.int32),
            pltpu.VMEM((_B, _L), jnp.float32),
            pltpu.VMEM((_ZR, _L), jnp.float32),
            pltpu.VMEM_SHARED((_NPAD, _L), jnp.float32),
        ],
    )
    def count_k(edge_hbm, out_hbm, didx, ones, zbuf, acc):
        c = lax.axis_index("c")
        s = lax.axis_index("s")
        one16 = jnp.ones((_L,), jnp.float32)

        @pl.loop(0, _B)
        def _(i):
            ones[i, :] = one16

        _zero_acc(zbuf, acc, s)
        plsc.subcore_barrier()

        w = s * _NC + c
        start, stop = _worker_span(w, nchunks, _NC * _NS)

        @pl.loop(start, stop)
        def _(ci):
            pltpu.sync_copy(edge_hbm.at[1, ci, :, :], didx)

            @pl.loop(0, _K)
            def _(k):
                pltpu.sync_copy(ones, acc.at[didx.at[k]], add=True)

        plsc.subcore_barrier()
        _flush_acc(acc, out_hbm, c, s)

    return count_k(edge4)


def _sc_scatter_pass(table, edge4, n_nodes, feature_split):
    nchunks = edge4.shape[1]
    mesh = plsc.VectorSubcoreMesh(core_axis_name="c", subcore_axis_name="s")

    @functools.partial(
        pl.kernel,
        out_type=jax.ShapeDtypeStruct((_NC, _NPAD, _L), jnp.float32),
        mesh=mesh,
        compiler_params=_SC_PARAMS,
        scratch_types=[
            pltpu.VMEM((_K, _B), jnp.int32),
            pltpu.VMEM((_K, _B), jnp.int32),
            pltpu.VMEM((_D, _B, _L), jnp.float32),
            pltpu.VMEM((_ZR, _L), jnp.float32),
            pltpu.VMEM_SHARED((_NPAD, _L), jnp.float32),
            pltpu.SemaphoreType.DMA((_D,)),
        ],
    )
    def pass_k(table_hbm, edge_hbm, out_hbm, sidx, didx, rows, zbuf, acc, gsem):
        c = lax.axis_index("c")
        s = lax.axis_index("s")

        _zero_acc(zbuf, acc, s)
        plsc.subcore_barrier()

        if feature_split:
            w = s
            start, stop = _worker_span(w, nchunks, _NS)
        else:
            w = s * _NC + c
            start, stop = _worker_span(w, nchunks, _NC * _NS)

        @pl.loop(start, stop)
        def _(ci):
            pltpu.sync_copy(edge_hbm.at[0, ci, :, :], sidx)
            pltpu.sync_copy(edge_hbm.at[1, ci, :, :], didx)
            tbl = table_hbm.at[c] if feature_split else table_hbm

            for k in range(_D):
                pltpu.async_copy(tbl.at[sidx.at[k]], rows.at[k], gsem.at[k])

            @pl.loop(0, _K)
            def _(k):
                slot = lax.rem(k, _D)
                pltpu.make_async_copy(
                    tbl.at[sidx.at[k]], rows.at[slot], gsem.at[slot]
                ).wait()
                pltpu.sync_copy(rows.at[slot], acc.at[didx.at[k]], add=True)

                @pl.when(k + _D < _K)
                def _():
                    pltpu.async_copy(
                        tbl.at[sidx.at[k + _D]], rows.at[slot], gsem.at[slot]
                    )

        plsc.subcore_barrier()
        _flush_acc(acc, out_hbm, c, s)

    return pass_k(table, edge4)


def _tc_prep(degp, x):
    n, in_dim = x.shape

    def body(degp_ref, x_ref, dinv_ref, g1_ref):
        d = degp_ref[0, :, 0] + degp_ref[1, :, 0] + 1.0
        di = lax.rsqrt(d)[:, None]
        dinv_ref[...] = jnp.broadcast_to(di, (_BN, _L))
        xd = x_ref[...] * di
        g1_ref[...] = jnp.concatenate(
            [xd, jnp.zeros((_BN, _L - in_dim), jnp.float32)], axis=1
        )

    return pl.pallas_call(
        body,
        grid=(n // _BN,),
        in_specs=[
            pl.BlockSpec((_NC, _BN, _L), lambda j: (0, j, 0)),
            pl.BlockSpec((_BN, in_dim), lambda j: (j, 0)),
        ],
        out_specs=[
            pl.BlockSpec((_BN, _L), lambda j: (j, 0)),
            pl.BlockSpec((_BN, _L), lambda j: (j, 0)),
        ],
        out_shape=[
            jax.ShapeDtypeStruct((n, _L), jnp.float32),
            jax.ShapeDtypeStruct((n, _L), jnp.float32),
        ],
    )(degp, x)


def _tc_layer1(pB, g1, dinv, W1, b1):
    n = g1.shape[0]
    in_dim, hid = W1.shape

    def body(p_ref, g1_ref, dinv_ref, w_ref, b_ref, t2_ref):
        di = dinv_ref[...]
        sacc = (p_ref[0] + p_ref[1] + g1_ref[...]) * di
        s3 = sacc[:, :in_dim]
        h1 = jax.lax.dot_general(
            s3, w_ref[...], (((1,), (0,)), ((), ())),
            preferred_element_type=jnp.float32,
        )
        h1 = jnp.maximum(h1 + b_ref[...], 0.0)
        t2_ref[0] = h1[:, :_L] * di
        t2_ref[1] = h1[:, _L:] * di

    return pl.pallas_call(
        body,
        grid=(n // _BN,),
        in_specs=[
            pl.BlockSpec((_NC, _BN, _L), lambda j: (0, j, 0)),
            pl.BlockSpec((_BN, _L), lambda j: (j, 0)),
            pl.BlockSpec((_BN, _L), lambda j: (j, 0)),
            pl.BlockSpec((in_dim, hid), lambda j: (0, 0)),
            pl.BlockSpec((1, hid), lambda j: (0, 0)),
        ],
        out_specs=[pl.BlockSpec((_NC, _BN, _L), lambda j: (0, j, 0))],
        out_shape=[jax.ShapeDtypeStruct((_NC, n, _L), jnp.float32)],
    )(pB, g1, dinv, W1, b1)[0]


def _tc_layer2(pC, t2, dinv, W2, b2, W3):
    n = dinv.shape[0]
    hid = W2.shape[0]
    lat = W3.shape[1]

    def body(p_ref, t2_ref, dinv_ref, w2_ref, b2_ref, w3_ref, g3_ref):
        di = dinv_ref[...]
        o2a = (p_ref[0] + t2_ref[0]) * di
        o2b = (p_ref[1] + t2_ref[1]) * di
        out2 = jnp.concatenate([o2a, o2b], axis=1)
        h2 = jax.lax.dot_general(
            out2, w2_ref[...], (((1,), (0,)), ((), ())),
            preferred_element_type=jnp.float32,
        )
        h2 = jnp.maximum(h2 + b2_ref[...], 0.0)
        t = jax.lax.dot_general(
            h2, w3_ref[...], (((1,), (0,)), ((), ())),
            preferred_element_type=jnp.float32,
        )
        g3_ref[...] = jnp.concatenate(
            [t * di[:, :lat], jnp.zeros((_BN, _L - lat), jnp.float32)], axis=1
        )

    return pl.pallas_call(
        body,
        grid=(n // _BN,),
        in_specs=[
            pl.BlockSpec((_NC, _BN, _L), lambda j: (0, j, 0)),
            pl.BlockSpec((_NC, _BN, _L), lambda j: (0, j, 0)),
            pl.BlockSpec((_BN, _L), lambda j: (j, 0)),
            pl.BlockSpec((hid, hid), lambda j: (0, 0)),
            pl.BlockSpec((1, hid), lambda j: (0, 0)),
            pl.BlockSpec((hid, lat), lambda j: (0, 0)),
        ],
        out_specs=[pl.BlockSpec((_BN, _L), lambda j: (j, 0))],
        out_shape=[jax.ShapeDtypeStruct((n, _L), jnp.float32)],
    )(pC, t2, dinv, W2, b2, W3)[0]


def _tc_layer3(pD, g3, dinv, b3):
    n = dinv.shape[0]
    lat = b3.shape[1]

    def body(p_ref, g3_ref, dinv_ref, b_ref, out_ref):
        sacc = (p_ref[0] + p_ref[1] + g3_ref[...]) * dinv_ref[...]
        out_ref[...] = sacc[:, :lat] + b_ref[...]

    return pl.pallas_call(
        body,
        grid=(n // _BN,),
        in_specs=[
            pl.BlockSpec((_NC, _BN, _L), lambda j: (0, j, 0)),
            pl.BlockSpec((_BN, _L), lambda j: (j, 0)),
            pl.BlockSpec((_BN, _L), lambda j: (j, 0)),
            pl.BlockSpec((1, lat), lambda j: (0, 0)),
        ],
        out_specs=[pl.BlockSpec((_BN, lat), lambda j: (j, 0))],
        out_shape=[jax.ShapeDtypeStruct((n, lat), jnp.float32)],
    )(pD, g3, dinv, b3)[0]


def kernel(x, edge_index, W1, b1, W2, b2, W3, b3):
    n = x.shape[0]
    e = edge_index.shape[1]
    edge4 = edge_index.reshape(2, e // (_K * _B), _K, _B)

    degp = _sc_count(edge4, n)
    dinv, g1 = _tc_prep(degp, x)
    pB = _sc_scatter_pass(g1, edge4, n, False)
    t2 = _tc_layer1(pB, g1, dinv, W1, b1.reshape(1, -1))
    pC = _sc_scatter_pass(t2, edge4, n, True)
    g3 = _tc_layer2(pC, t2, dinv, W2, b2.reshape(1, -1), W3)
    pD = _sc_scatter_pass(g3, edge4, n, False)
    out = _tc_layer3(pD, g3, dinv, b3.reshape(1, -1))
    return out
